# baseline (speedup 1.0000x reference)
import functools

import jax
import jax.numpy as jnp
from jax import lax
from jax.experimental import pallas as pl
from jax.experimental.pallas import tpu as pltpu
from jax.experimental.pallas import tpu_sc as plsc

N = 100000
D = 119
DP = 128
L = 16
NC, NS = 2, 16
NW = NC * NS
GROUPS = 16
BLOCK = GROUPS * L
BT = BLOCK // 8
NB = 3
ZRING = 6

NBLK = N // BLOCK
TFULL = NBLK // NW
XBLK = NBLK - TFULL * NW
KMAX = TFULL + 1
TAIL0 = NBLK * BLOCK
TAILG = (N - TAIL0) // L


@functools.cache
def _build_sc():
    mesh = plsc.VectorSubcoreMesh(
        core_axis_name="c", subcore_axis_name="s", num_cores=NC, num_subcores=NS
    )

    @functools.partial(
        pl.kernel,
        out_type=jax.ShapeDtypeStruct((N, DP), jnp.int32),
        mesh=mesh,
        compiler_params=pltpu.CompilerParams(needs_layout_passes=False),
        scratch_types=[
            pltpu.VMEM((128,), jnp.int32),
            pltpu.VMEM((ZRING, BLOCK), jnp.int32),
            pltpu.VMEM((NB, BT, 8, DP), jnp.int32),
        ]
        + [pltpu.SemaphoreType.DMA] * (ZRING + NB),
    )
    def onehot_sc(z_hbm, table_hbm, out_hbm, table_v, z_v, buf, *sems):
        zsems = sems[:ZRING]
        osems = sems[ZRING:]
        out3 = out_hbm.reshape(N // 8, 8, DP)
        wid = lax.axis_index("s") * NC + lax.axis_index("c")
        iota = lax.iota(jnp.int32, L)
        ones = jnp.ones((L,), jnp.int32)
        zeros = jnp.zeros((L,), jnp.int32)
        sub = jnp.bitwise_and(iota, 7)
        tof = lax.shift_right_logical(iota, 3)
        in_x = wid < XBLK

        def row0_of(k):
            return (k * NW + wid) * BLOCK

        def start_z(k):
            pltpu.async_copy(
                z_hbm.at[pl.ds(row0_of(k), BLOCK)],
                z_v.at[k % ZRING],
                zsems[k % ZRING],
            )

        def wait_z(k):
            pltpu.make_async_copy(
                z_hbm.at[pl.ds(row0_of(k), BLOCK)],
                z_v.at[k % ZRING],
                zsems[k % ZRING],
            ).wait()

        def start_out(k):
            pltpu.async_copy(
                buf.at[k % NB],
                out3.at[pl.ds(row0_of(k) // 8, BT)],
                osems[k % NB],
            )

        def wait_out(k):
            pltpu.make_async_copy(
                buf.at[k % NB],
                out3.at[pl.ds(row0_of(k) // 8, BT)],
                osems[k % NB],
            ).wait()

        def scatter_vals(bank, slot, ngroups, vals):
            bvec = jnp.full((L,), bank, jnp.int32)
            for g in range(ngroups):
                zv = z_v[slot, pl.ds(g * L, L)]
                idx = plsc.load_gather(table_v, [zv])
                plsc.store_scatter(buf, [bvec, 2 * g + tof, sub, idx], vals)

        pltpu.sync_copy(table_hbm, table_v)

        def _zero_tile(t, carry):
            for b in range(NB):
                for s in range(8):
                    for g in range(8):
                        buf[b, t, s, pl.ds(g * L, L)] = zeros
            return carry

        lax.fori_loop(0, BT, _zero_tile, 0)

        for k in range(NB):
            start_z(k)

        def step(k):
            if k >= NB:
                wait_out(k - NB)
                scatter_vals(k % NB, (k - NB) % ZRING, GROUPS, zeros)
            wait_z(k)
            scatter_vals(k % NB, k % ZRING, GROUPS, ones)
            start_out(k)
            if k + NB < TFULL:
                start_z(k + NB)
            elif k + NB == TFULL:
                @pl.when(in_x)
                def _():
                    start_z(k + NB)

        for k in range(KMAX):
            if k < TFULL:
                step(k)
            else:
                @pl.when(in_x)
                def _():
                    step(k)

        for k in range(TFULL - NB, TFULL):
            wait_out(k)

        @pl.when(wid < TAILG)
        def _():
            trow = TAIL0 + wid * L
            for t in range(2):
                for s in range(8):
                    for g in range(8):
                        buf[0, t, s, pl.ds(g * L, L)] = zeros
            pltpu.sync_copy(
                z_hbm.at[pl.ds(trow, L)], z_v.at[0, pl.ds(0, L)]
            )
            zv = z_v[0, pl.ds(0, L)]
            idx = plsc.load_gather(table_v, [zv])
            plsc.store_scatter(
                buf, [jnp.zeros((L,), jnp.int32), tof, sub, idx], ones
            )
            pltpu.sync_copy(
                buf.at[0, pl.ds(0, 2)], out3.at[pl.ds(trow // 8, 2)]
            )

    return onehot_sc


def kernel(Z, z_to_index):
    zi = Z.astype(jnp.int32)
    table = jnp.zeros((128,), jnp.int32).at[:D].set(z_to_index.astype(jnp.int32))
    padded = _build_sc()(zi, table)
    return padded[:, :D]
